# baseline (speedup 1.0000x reference)
import functools

import jax
import jax.numpy as jnp
from jax import lax
from jax.experimental import pallas as pl
from jax.experimental.pallas import tpu as pltpu
from jax.experimental.pallas import tpu_sc as plsc

N = 10000
E = 160000
D_IN = 256
H = 512
HF = 256
NCLS = 10
NLAYERS = 4
G = 16

NB = 1000
NBLK = N // NB

C = 4
F = H // C
NC = 2
NS = 16
EPT = E // NS
EB = 128
NBAT = 79
EPAD = NBAT * EB
PAD = EPAD - EPT
NP = N + 8
SEGS = ((0, 40), (40, 39))
SRCBUF = 40 * EB
RB = 624
TAIL = N - NS * RB
ZR = 104

_F32 = jnp.float32


def _tc_init_body(x_ref, wini_ref, bini_ref, wm0_ref, bm0_ref, batch_ref,
                  um_ref, psumx_ref, counts_ref):
    i = pl.program_id(0)
    x = x_ref[...]
    u0 = jnp.dot(x, wini_ref[...], preferred_element_type=_F32) + bini_ref[...]
    um = jnp.dot(u0, wm0_ref[...], preferred_element_type=_F32) + bm0_ref[...]
    for c in range(C):
        um_ref[c] = um[:, c * F:(c + 1) * F]
    b = batch_ref[...].reshape(1, NB)
    oh = (lax.broadcasted_iota(jnp.int32, (G, NB), 0) == b).astype(_F32)

    @pl.when(i == 0)
    def _():
        psumx_ref[...] = jnp.zeros_like(psumx_ref)
        counts_ref[...] = jnp.zeros_like(counts_ref)

    psumx_ref[...] += jnp.dot(oh, x, preferred_element_type=_F32)
    counts_ref[...] += jnp.broadcast_to(
        jnp.sum(oh, axis=1, keepdims=True), (G, H))


def _tc_init(x, w_init, b_init, wm0, bm0, batch3):
    return pl.pallas_call(
        _tc_init_body,
        grid=(NBLK,),
        in_specs=[
            pl.BlockSpec((NB, D_IN), lambda i: (i, 0)),
            pl.BlockSpec((D_IN, H), lambda i: (0, 0)),
            pl.BlockSpec((1, H), lambda i: (0, 0)),
            pl.BlockSpec((H, H), lambda i: (0, 0)),
            pl.BlockSpec((1, H), lambda i: (0, 0)),
            pl.BlockSpec((1, 1, NB), lambda i: (i, 0, 0)),
        ],
        out_specs=[
            pl.BlockSpec((C, NB, F), lambda i: (0, i, 0)),
            pl.BlockSpec((G, D_IN), lambda i: (0, 0)),
            pl.BlockSpec((G, H), lambda i: (0, 0)),
        ],
        out_shape=[
            jax.ShapeDtypeStruct((C, N, F), _F32),
            jax.ShapeDtypeStruct((G, D_IN), _F32),
            jax.ShapeDtypeStruct((G, H), _F32),
        ],
        compiler_params=pltpu.CompilerParams(
            dimension_semantics=("arbitrary",)),
    )(x, w_init, b_init, wm0, bm0, batch3)


def _tc_update_body(aggr_ref, um_ref, wi_ref, bi_ref, wj_ref, bj_ref,
                    wn_ref, bn_ref, batch_ref, umn_ref, psum_ref):
    i = pl.program_id(0)
    aggr = jnp.concatenate([aggr_ref[c] for c in range(C)], axis=-1)
    um = jnp.concatenate([um_ref[c] for c in range(C)], axis=-1)
    ai = um * wi_ref[...] + bi_ref[...]
    aj = aggr * wj_ref[...] + bj_ref[...]
    u = aggr + um + ai * aj
    umn = jnp.dot(u, wn_ref[...], preferred_element_type=_F32) + bn_ref[...]
    for c in range(C):
        umn_ref[c] = umn[:, c * F:(c + 1) * F]
    b = batch_ref[...].reshape(1, NB)
    oh = (lax.broadcasted_iota(jnp.int32, (G, NB), 0) == b).astype(_F32)

    @pl.when(i == 0)
    def _():
        psum_ref[...] = jnp.zeros_like(psum_ref)

    psum_ref[...] += jnp.dot(oh, u, preferred_element_type=_F32)


def _tc_update(aggr, um, wi, bi, wj, bj, wn, bn, batch3):
    return pl.pallas_call(
        _tc_update_body,
        grid=(NBLK,),
        in_specs=[
            pl.BlockSpec((C, NB, F), lambda i: (0, i, 0)),
            pl.BlockSpec((C, NB, F), lambda i: (0, i, 0)),
            pl.BlockSpec((1, H), lambda i: (0, 0)),
            pl.BlockSpec((1, H), lambda i: (0, 0)),
            pl.BlockSpec((1, H), lambda i: (0, 0)),
            pl.BlockSpec((1, H), lambda i: (0, 0)),
            pl.BlockSpec((H, H), lambda i: (0, 0)),
            pl.BlockSpec((1, H), lambda i: (0, 0)),
            pl.BlockSpec((1, 1, NB), lambda i: (i, 0, 0)),
        ],
        out_specs=[
            pl.BlockSpec((C, NB, F), lambda i: (0, i, 0)),
            pl.BlockSpec((G, H), lambda i: (0, 0)),
        ],
        out_shape=[
            jax.ShapeDtypeStruct((C, N, F), _F32),
            jax.ShapeDtypeStruct((G, H), _F32),
        ],
        compiler_params=pltpu.CompilerParams(
            dimension_semantics=("arbitrary",)),
    )(aggr, um, wi, bi, wj, bj, wn, bn, batch3)


def _tc_last_body(aggr_ref, um_ref, wi_ref, bi_ref, wj_ref, bj_ref,
                  batch_ref, psum_ref):
    i = pl.program_id(0)
    aggr = jnp.concatenate([aggr_ref[c] for c in range(C)], axis=-1)
    um = jnp.concatenate([um_ref[c] for c in range(C)], axis=-1)
    ai = um * wi_ref[...] + bi_ref[...]
    aj = aggr * wj_ref[...] + bj_ref[...]
    u = aggr + um + ai * aj
    b = batch_ref[...].reshape(1, NB)
    oh = (lax.broadcasted_iota(jnp.int32, (G, NB), 0) == b).astype(_F32)

    @pl.when(i == 0)
    def _():
        psum_ref[...] = jnp.zeros_like(psum_ref)

    psum_ref[...] += jnp.dot(oh, u, preferred_element_type=_F32)


def _tc_last(aggr, um, wi, bi, wj, bj, batch3):
    return pl.pallas_call(
        _tc_last_body,
        grid=(NBLK,),
        in_specs=[
            pl.BlockSpec((C, NB, F), lambda i: (0, i, 0)),
            pl.BlockSpec((C, NB, F), lambda i: (0, i, 0)),
            pl.BlockSpec((1, H), lambda i: (0, 0)),
            pl.BlockSpec((1, H), lambda i: (0, 0)),
            pl.BlockSpec((1, H), lambda i: (0, 0)),
            pl.BlockSpec((1, H), lambda i: (0, 0)),
            pl.BlockSpec((1, 1, NB), lambda i: (i, 0, 0)),
        ],
        out_specs=pl.BlockSpec((G, H), lambda i: (0, 0)),
        out_shape=jax.ShapeDtypeStruct((G, H), _F32),
        compiler_params=pltpu.CompilerParams(
            dimension_semantics=("arbitrary",)),
    )(aggr, um, wi, bi, wj, bj, batch3)


def _tc_head_body(psumx_ref, ps0_ref, ps1_ref, ps2_ref, ps3_ref, counts_ref,
                  wnp_ref, bnp_ref, we_ref, be_ref, wf_ref, bf_ref, out_ref):
    cnt = jnp.maximum(counts_ref[...], 1.0)
    poolx = psumx_ref[...] / cnt[:, :D_IN]
    out = jnp.dot(poolx, wnp_ref[...], preferred_element_type=_F32) + bnp_ref[...]
    for l, ps_ref in enumerate((ps0_ref, ps1_ref, ps2_ref, ps3_ref)):
        pool = ps_ref[...] / cnt
        out += (jnp.dot(pool, we_ref[l], preferred_element_type=_F32)
                + be_ref[l]) * (1.0 / NLAYERS)
    logits = jnp.dot(out, wf_ref[...], preferred_element_type=_F32) + bf_ref[...]
    m = jnp.max(logits, axis=-1, keepdims=True)
    s = logits - m
    lse = jnp.log(jnp.sum(jnp.exp(s), axis=-1, keepdims=True))
    out_ref[...] = s - lse


def _tc_head(psumx, psums, counts, w_np, b_np, we, be, wf, bf):
    return pl.pallas_call(
        _tc_head_body,
        out_shape=jax.ShapeDtypeStruct((G, NCLS), _F32),
    )(psumx, psums[0], psums[1], psums[2], psums[3], counts,
      w_np, b_np, we, be, wf, bf)


def _sc_aggregate(um_flat, srcoff, dst3, zrows):
    mesh = plsc.VectorSubcoreMesh(core_axis_name="c", subcore_axis_name="s")

    @functools.partial(
        pl.kernel,
        out_type=jax.ShapeDtypeStruct((C * N, F), _F32),
        mesh=mesh,
        scratch_types=[
            pltpu.VMEM((SRCBUF,), jnp.int32),
            pltpu.VMEM((NBAT, EB), jnp.int32),
            pltpu.VMEM((EB, F), _F32),
            pltpu.VMEM((EB, F), _F32),
            pltpu.VMEM_SHARED((NP, F), _F32),
            pltpu.SemaphoreType.DMA,
            pltpu.SemaphoreType.DMA,
            pltpu.SemaphoreType.DMA,
            pltpu.SemaphoreType.DMA,
        ],
    )
    def body(um_hbm, src_hbm, dst_hbm, z_hbm, out_hbm,
             src_v, dst_v, rows_a, rows_b, acc_sh, gsa, gsb, ssa, ssb):
        cid = lax.axis_index("c")
        sid = lax.axis_index("s")
        pltpu.sync_copy(dst_hbm.at[sid], dst_v)

        def lidx(l):
            return src_v.at[pl.ds(pl.multiple_of(l * EB, 8), EB)]

        def run_segment(chunk, seg_start, nb):
            pltpu.sync_copy(
                src_hbm.at[chunk, sid, pl.ds(seg_start * EB, nb * EB)],
                src_v.at[pl.ds(0, nb * EB)])
            pltpu.async_copy(um_hbm.at[lidx(0)], rows_a, gsa)

            def pair(i, carry):
                l = i * 2
                b = seg_start + l
                pltpu.make_async_copy(um_hbm.at[lidx(l)], rows_a, gsa).wait()

                @pl.when(l > 0)
                def _():
                    pltpu.make_async_copy(
                        rows_b, acc_sh.at[dst_v.at[b]], ssb).wait()

                pltpu.async_copy(um_hbm.at[lidx(l + 1)], rows_b, gsb)
                pltpu.async_copy(rows_a, acc_sh.at[dst_v.at[b]], ssa,
                                 add=True)
                pltpu.make_async_copy(
                    um_hbm.at[lidx(l + 1)], rows_b, gsb).wait()
                pltpu.make_async_copy(
                    rows_a, acc_sh.at[dst_v.at[b]], ssa).wait()

                @pl.when(l + 2 < nb)
                def _():
                    pltpu.async_copy(um_hbm.at[lidx(l + 2)], rows_a, gsa)

                pltpu.async_copy(rows_b, acc_sh.at[dst_v.at[b + 1]], ssb,
                                 add=True)
                return carry

            lax.fori_loop(0, nb // 2, pair, 0)
            if nb % 2 == 1:
                l = nb - 1
                b = seg_start + l
                pltpu.make_async_copy(um_hbm.at[lidx(l)], rows_a, gsa).wait()
                pltpu.make_async_copy(
                    rows_b, acc_sh.at[dst_v.at[b - 1]], ssb).wait()
                pltpu.async_copy(rows_a, acc_sh.at[dst_v.at[b]], ssa,
                                 add=True)
                pltpu.make_async_copy(
                    rows_a, acc_sh.at[dst_v.at[b]], ssa).wait()
            else:
                pltpu.make_async_copy(
                    rows_b, acc_sh.at[dst_v.at[seg_start]], ssb).wait()

        for c_local in range(NC):
            chunk = cid * NC + c_local

            for z in range(RB // ZR):
                pltpu.sync_copy(
                    z_hbm, acc_sh.at[pl.ds(sid * RB + z * ZR, ZR)])

            @pl.when(sid == 0)
            def _():
                pltpu.sync_copy(z_hbm.at[pl.ds(0, TAIL)],
                                acc_sh.at[pl.ds(NS * RB, TAIL)])

            plsc.subcore_barrier()

            for seg_start, nb in SEGS:
                run_segment(chunk, seg_start, nb)
            plsc.subcore_barrier()

            pltpu.sync_copy(
                acc_sh.at[pl.ds(sid * RB, RB)],
                out_hbm.at[pl.ds(chunk * N + sid * RB, RB)])

            @pl.when(sid == 0)
            def _():
                pltpu.sync_copy(
                    acc_sh.at[pl.ds(NS * RB, TAIL)],
                    out_hbm.at[pl.ds(chunk * N + NS * RB, TAIL)])

            plsc.subcore_barrier()

    return body(um_flat, srcoff, dst3, zrows)


def kernel(x, edge_index, batch, W_init, b_init, W_np, b_np,
           Wm, bm, wi, bi, wj, bj, We, be, Wf, bf):
    srcp = jnp.concatenate(
        [edge_index[0].reshape(NS, EPT),
         jnp.zeros((NS, PAD), jnp.int32)], axis=1)
    srcoff = (srcp[None]
              + (jnp.arange(C, dtype=jnp.int32) * N)[:, None, None])
    dst3 = jnp.concatenate(
        [edge_index[1].reshape(NS, EPT),
         jnp.full((NS, PAD), N, jnp.int32)], axis=1).reshape(NS, NBAT, EB)
    batch3 = batch.reshape(NBLK, 1, NB)
    zrows = jnp.zeros((ZR, F), _F32)

    b_init2 = b_init.reshape(1, H)
    bm2 = bm.reshape(NLAYERS, 1, H)
    wi2 = wi.reshape(NLAYERS, 1, H)
    bi2 = bi.reshape(NLAYERS, 1, H)
    wj2 = wj.reshape(NLAYERS, 1, H)
    bj2 = bj.reshape(NLAYERS, 1, H)
    be2 = be.reshape(NLAYERS, 1, HF)
    b_np2 = b_np.reshape(1, HF)
    bf2 = bf.reshape(1, NCLS)

    um, psumx, counts = _tc_init(x, W_init, b_init2, Wm[0], bm2[0], batch3)
    psums = []
    for l in range(NLAYERS):
        aggr = _sc_aggregate(um.reshape(C * N, F), srcoff, dst3, zrows)
        aggr = aggr.reshape(C, N, F)
        if l < NLAYERS - 1:
            um, ps = _tc_update(aggr, um, wi2[l], bi2[l], wj2[l], bj2[l],
                                Wm[l + 1], bm2[l + 1], batch3)
        else:
            ps = _tc_last(aggr, um, wi2[l], bi2[l], wj2[l], bj2[l], batch3)
        psums.append(ps)

    return _tc_head(psumx, psums, counts, W_np, b_np2, We, be2, Wf, bf2)


# speedup vs baseline: 1.0034x; 1.0034x over previous
import functools

import jax
import jax.numpy as jnp
from jax import lax
from jax.experimental import pallas as pl
from jax.experimental.pallas import tpu as pltpu
from jax.experimental.pallas import tpu_sc as plsc

N = 10000
E = 160000
D_IN = 256
H = 512
HF = 256
NCLS = 10
NLAYERS = 4
G = 16

NB = 1000
NBLK = N // NB

C = 4
F = H // C
NC = 2
NS = 16
EPT = E // NS
EB = 128
NBAT = 79
EPAD = NBAT * EB
PAD = EPAD - EPT
NP = N + NS
SEGS = ((0, 40), (40, 39))
SRCBUF = 40 * EB
RB = 624
TAIL = N - NS * RB
ZR = 104

_F32 = jnp.float32


def _tc_init_body(x_ref, wini_ref, bini_ref, wm0_ref, bm0_ref, batch_ref,
                  um_ref, psumx_ref, counts_ref):
    i = pl.program_id(0)
    x = x_ref[...]
    u0 = jnp.dot(x, wini_ref[...], preferred_element_type=_F32) + bini_ref[...]
    um = jnp.dot(u0, wm0_ref[...], preferred_element_type=_F32) + bm0_ref[...]
    for c in range(C):
        um_ref[c] = um[:, c * F:(c + 1) * F]
    b = batch_ref[...].reshape(1, NB)
    oh = (lax.broadcasted_iota(jnp.int32, (G, NB), 0) == b).astype(_F32)

    @pl.when(i == 0)
    def _():
        psumx_ref[...] = jnp.zeros_like(psumx_ref)
        counts_ref[...] = jnp.zeros_like(counts_ref)

    psumx_ref[...] += jnp.dot(oh, x, preferred_element_type=_F32)
    counts_ref[...] += jnp.broadcast_to(
        jnp.sum(oh, axis=1, keepdims=True), (G, H))


def _tc_init(x, w_init, b_init, wm0, bm0, batch3):
    return pl.pallas_call(
        _tc_init_body,
        grid=(NBLK,),
        in_specs=[
            pl.BlockSpec((NB, D_IN), lambda i: (i, 0)),
            pl.BlockSpec((D_IN, H), lambda i: (0, 0)),
            pl.BlockSpec((1, H), lambda i: (0, 0)),
            pl.BlockSpec((H, H), lambda i: (0, 0)),
            pl.BlockSpec((1, H), lambda i: (0, 0)),
            pl.BlockSpec((1, 1, NB), lambda i: (i, 0, 0)),
        ],
        out_specs=[
            pl.BlockSpec((C, NB, F), lambda i: (0, i, 0)),
            pl.BlockSpec((G, D_IN), lambda i: (0, 0)),
            pl.BlockSpec((G, H), lambda i: (0, 0)),
        ],
        out_shape=[
            jax.ShapeDtypeStruct((C, N, F), _F32),
            jax.ShapeDtypeStruct((G, D_IN), _F32),
            jax.ShapeDtypeStruct((G, H), _F32),
        ],
        compiler_params=pltpu.CompilerParams(
            dimension_semantics=("arbitrary",)),
    )(x, w_init, b_init, wm0, bm0, batch3)


def _tc_update_body(aggr_ref, um_ref, wi_ref, bi_ref, wj_ref, bj_ref,
                    wn_ref, bn_ref, batch_ref, umn_ref, psum_ref):
    i = pl.program_id(0)
    aggr = jnp.concatenate([aggr_ref[c] for c in range(C)], axis=-1)
    um = jnp.concatenate([um_ref[c] for c in range(C)], axis=-1)
    ai = um * wi_ref[...] + bi_ref[...]
    aj = aggr * wj_ref[...] + bj_ref[...]
    u = aggr + um + ai * aj
    umn = jnp.dot(u, wn_ref[...], preferred_element_type=_F32) + bn_ref[...]
    for c in range(C):
        umn_ref[c] = umn[:, c * F:(c + 1) * F]
    b = batch_ref[...].reshape(1, NB)
    oh = (lax.broadcasted_iota(jnp.int32, (G, NB), 0) == b).astype(_F32)

    @pl.when(i == 0)
    def _():
        psum_ref[...] = jnp.zeros_like(psum_ref)

    psum_ref[...] += jnp.dot(oh, u, preferred_element_type=_F32)


def _tc_update(aggr, um, wi, bi, wj, bj, wn, bn, batch3):
    return pl.pallas_call(
        _tc_update_body,
        grid=(NBLK,),
        in_specs=[
            pl.BlockSpec((C, NB, F), lambda i: (0, i, 0)),
            pl.BlockSpec((C, NB, F), lambda i: (0, i, 0)),
            pl.BlockSpec((1, H), lambda i: (0, 0)),
            pl.BlockSpec((1, H), lambda i: (0, 0)),
            pl.BlockSpec((1, H), lambda i: (0, 0)),
            pl.BlockSpec((1, H), lambda i: (0, 0)),
            pl.BlockSpec((H, H), lambda i: (0, 0)),
            pl.BlockSpec((1, H), lambda i: (0, 0)),
            pl.BlockSpec((1, 1, NB), lambda i: (i, 0, 0)),
        ],
        out_specs=[
            pl.BlockSpec((C, NB, F), lambda i: (0, i, 0)),
            pl.BlockSpec((G, H), lambda i: (0, 0)),
        ],
        out_shape=[
            jax.ShapeDtypeStruct((C, N, F), _F32),
            jax.ShapeDtypeStruct((G, H), _F32),
        ],
        compiler_params=pltpu.CompilerParams(
            dimension_semantics=("arbitrary",)),
    )(aggr, um, wi, bi, wj, bj, wn, bn, batch3)


def _tc_last_body(aggr_ref, um_ref, wi_ref, bi_ref, wj_ref, bj_ref,
                  batch_ref, psum_ref):
    i = pl.program_id(0)
    aggr = jnp.concatenate([aggr_ref[c] for c in range(C)], axis=-1)
    um = jnp.concatenate([um_ref[c] for c in range(C)], axis=-1)
    ai = um * wi_ref[...] + bi_ref[...]
    aj = aggr * wj_ref[...] + bj_ref[...]
    u = aggr + um + ai * aj
    b = batch_ref[...].reshape(1, NB)
    oh = (lax.broadcasted_iota(jnp.int32, (G, NB), 0) == b).astype(_F32)

    @pl.when(i == 0)
    def _():
        psum_ref[...] = jnp.zeros_like(psum_ref)

    psum_ref[...] += jnp.dot(oh, u, preferred_element_type=_F32)


def _tc_last(aggr, um, wi, bi, wj, bj, batch3):
    return pl.pallas_call(
        _tc_last_body,
        grid=(NBLK,),
        in_specs=[
            pl.BlockSpec((C, NB, F), lambda i: (0, i, 0)),
            pl.BlockSpec((C, NB, F), lambda i: (0, i, 0)),
            pl.BlockSpec((1, H), lambda i: (0, 0)),
            pl.BlockSpec((1, H), lambda i: (0, 0)),
            pl.BlockSpec((1, H), lambda i: (0, 0)),
            pl.BlockSpec((1, H), lambda i: (0, 0)),
            pl.BlockSpec((1, 1, NB), lambda i: (i, 0, 0)),
        ],
        out_specs=pl.BlockSpec((G, H), lambda i: (0, 0)),
        out_shape=jax.ShapeDtypeStruct((G, H), _F32),
        compiler_params=pltpu.CompilerParams(
            dimension_semantics=("arbitrary",)),
    )(aggr, um, wi, bi, wj, bj, batch3)


def _tc_head_body(psumx_ref, ps0_ref, ps1_ref, ps2_ref, ps3_ref, counts_ref,
                  wnp_ref, bnp_ref, we_ref, be_ref, wf_ref, bf_ref, out_ref):
    cnt = jnp.maximum(counts_ref[...], 1.0)
    poolx = psumx_ref[...] / cnt[:, :D_IN]
    out = jnp.dot(poolx, wnp_ref[...], preferred_element_type=_F32) + bnp_ref[...]
    for l, ps_ref in enumerate((ps0_ref, ps1_ref, ps2_ref, ps3_ref)):
        pool = ps_ref[...] / cnt
        out += (jnp.dot(pool, we_ref[l], preferred_element_type=_F32)
                + be_ref[l]) * (1.0 / NLAYERS)
    logits = jnp.dot(out, wf_ref[...], preferred_element_type=_F32) + bf_ref[...]
    m = jnp.max(logits, axis=-1, keepdims=True)
    s = logits - m
    lse = jnp.log(jnp.sum(jnp.exp(s), axis=-1, keepdims=True))
    out_ref[...] = s - lse


def _tc_head(psumx, psums, counts, w_np, b_np, we, be, wf, bf):
    return pl.pallas_call(
        _tc_head_body,
        out_shape=jax.ShapeDtypeStruct((G, NCLS), _F32),
    )(psumx, psums[0], psums[1], psums[2], psums[3], counts,
      w_np, b_np, we, be, wf, bf)


def _sc_aggregate(um_flat, srcoff, dst3, zrows):
    mesh = plsc.VectorSubcoreMesh(core_axis_name="c", subcore_axis_name="s")

    @functools.partial(
        pl.kernel,
        out_type=jax.ShapeDtypeStruct((C * N, F), _F32),
        mesh=mesh,
        scratch_types=[
            pltpu.VMEM((SRCBUF,), jnp.int32),
            pltpu.VMEM((NBAT, EB), jnp.int32),
            pltpu.VMEM((EB, F), _F32),
            pltpu.VMEM((EB, F), _F32),
            pltpu.VMEM_SHARED((NP, F), _F32),
            pltpu.SemaphoreType.DMA,
            pltpu.SemaphoreType.DMA,
            pltpu.SemaphoreType.DMA,
            pltpu.SemaphoreType.DMA,
        ],
    )
    def body(um_hbm, src_hbm, dst_hbm, z_hbm, out_hbm,
             src_v, dst_v, rows_a, rows_b, acc_sh, gsa, gsb, ssa, ssb):
        cid = lax.axis_index("c")
        sid = lax.axis_index("s")
        pltpu.sync_copy(dst_hbm.at[sid], dst_v)

        def lidx(l):
            return src_v.at[pl.ds(pl.multiple_of(l * EB, 8), EB)]

        def run_segment(chunk, seg_start, nb):
            pltpu.sync_copy(
                src_hbm.at[chunk, sid, pl.ds(seg_start * EB, nb * EB)],
                src_v.at[pl.ds(0, nb * EB)])
            pltpu.async_copy(um_hbm.at[lidx(0)], rows_a, gsa)

            def pair(i, carry):
                l = i * 2
                b = seg_start + l
                pltpu.make_async_copy(um_hbm.at[lidx(l)], rows_a, gsa).wait()

                @pl.when(l > 0)
                def _():
                    pltpu.make_async_copy(
                        rows_b, acc_sh.at[dst_v.at[b]], ssb).wait()

                pltpu.async_copy(um_hbm.at[lidx(l + 1)], rows_b, gsb)
                pltpu.async_copy(rows_a, acc_sh.at[dst_v.at[b]], ssa,
                                 add=True)
                pltpu.make_async_copy(
                    um_hbm.at[lidx(l + 1)], rows_b, gsb).wait()
                pltpu.make_async_copy(
                    rows_a, acc_sh.at[dst_v.at[b]], ssa).wait()

                @pl.when(l + 2 < nb)
                def _():
                    pltpu.async_copy(um_hbm.at[lidx(l + 2)], rows_a, gsa)

                pltpu.async_copy(rows_b, acc_sh.at[dst_v.at[b + 1]], ssb,
                                 add=True)
                return carry

            lax.fori_loop(0, nb // 2, pair, 0)
            if nb % 2 == 1:
                l = nb - 1
                b = seg_start + l
                pltpu.make_async_copy(um_hbm.at[lidx(l)], rows_a, gsa).wait()
                pltpu.make_async_copy(
                    rows_b, acc_sh.at[dst_v.at[b - 1]], ssb).wait()
                pltpu.async_copy(rows_a, acc_sh.at[dst_v.at[b]], ssa,
                                 add=True)
                pltpu.make_async_copy(
                    rows_a, acc_sh.at[dst_v.at[b]], ssa).wait()
            else:
                pltpu.make_async_copy(
                    rows_b, acc_sh.at[dst_v.at[seg_start]], ssb).wait()

        for c_local in range(NC):
            chunk = cid * NC + c_local

            for z in range(RB // ZR):
                pltpu.sync_copy(
                    z_hbm, acc_sh.at[pl.ds(sid * RB + z * ZR, ZR)])

            @pl.when(sid == 0)
            def _():
                pltpu.sync_copy(z_hbm.at[pl.ds(0, TAIL)],
                                acc_sh.at[pl.ds(NS * RB, TAIL)])

            plsc.subcore_barrier()

            for seg_start, nb in SEGS:
                run_segment(chunk, seg_start, nb)
            plsc.subcore_barrier()

            pltpu.sync_copy(
                acc_sh.at[pl.ds(sid * RB, RB)],
                out_hbm.at[pl.ds(chunk * N + sid * RB, RB)])

            @pl.when(sid == 0)
            def _():
                pltpu.sync_copy(
                    acc_sh.at[pl.ds(NS * RB, TAIL)],
                    out_hbm.at[pl.ds(chunk * N + NS * RB, TAIL)])

            plsc.subcore_barrier()

    return body(um_flat, srcoff, dst3, zrows)


def kernel(x, edge_index, batch, W_init, b_init, W_np, b_np,
           Wm, bm, wi, bi, wj, bj, We, be, Wf, bf):
    srcp = jnp.concatenate(
        [edge_index[0].reshape(NS, EPT),
         jnp.zeros((NS, PAD), jnp.int32)], axis=1)
    srcoff = (srcp[None]
              + (jnp.arange(C, dtype=jnp.int32) * N)[:, None, None])
    dump = N + jnp.broadcast_to(
        jnp.arange(NS, dtype=jnp.int32)[:, None], (NS, PAD))
    dst3 = jnp.concatenate(
        [edge_index[1].reshape(NS, EPT), dump], axis=1).reshape(NS, NBAT, EB)
    batch3 = batch.reshape(NBLK, 1, NB)
    zrows = jnp.zeros((ZR, F), _F32)

    b_init2 = b_init.reshape(1, H)
    bm2 = bm.reshape(NLAYERS, 1, H)
    wi2 = wi.reshape(NLAYERS, 1, H)
    bi2 = bi.reshape(NLAYERS, 1, H)
    wj2 = wj.reshape(NLAYERS, 1, H)
    bj2 = bj.reshape(NLAYERS, 1, H)
    be2 = be.reshape(NLAYERS, 1, HF)
    b_np2 = b_np.reshape(1, HF)
    bf2 = bf.reshape(1, NCLS)

    um, psumx, counts = _tc_init(x, W_init, b_init2, Wm[0], bm2[0], batch3)
    psums = []
    for l in range(NLAYERS):
        aggr = _sc_aggregate(um.reshape(C * N, F), srcoff, dst3, zrows)
        aggr = aggr.reshape(C, N, F)
        if l < NLAYERS - 1:
            um, ps = _tc_update(aggr, um, wi2[l], bi2[l], wj2[l], bj2[l],
                                Wm[l + 1], bm2[l + 1], batch3)
        else:
            ps = _tc_last(aggr, um, wi2[l], bi2[l], wj2[l], bj2[l], batch3)
        psums.append(ps)

    return _tc_head(psumx, psums, counts, W_np, b_np2, We, be2, Wf, bf2)


# speedup vs baseline: 1.2168x; 1.2127x over previous
import functools

import jax
import jax.numpy as jnp
from jax import lax
from jax.experimental import pallas as pl
from jax.experimental.pallas import tpu as pltpu
from jax.experimental.pallas import tpu_sc as plsc

N = 10000
E = 160000
D_IN = 256
H = 512
HF = 256
NCLS = 10
NLAYERS = 4
G = 16

NB = 1000
NBLK = N // NB

C = 4
F = H // C
NC = 2
NS = 16
EPT = E // NS
EB = 80
NBAT = EPT // EB
EPAD = NBAT * EB
PAD = EPAD - EPT
NP = N
SEGS = ((0, NBAT),)
SRCBUF = NBAT * EB
RB = 624
TAIL = N - NS * RB
ZR = 104

_F32 = jnp.float32


def _tc_init_body(x_ref, wini_ref, bini_ref, wm0_ref, bm0_ref, batch_ref,
                  um_ref, psumx_ref, counts_ref):
    i = pl.program_id(0)
    x = x_ref[...]
    u0 = jnp.dot(x, wini_ref[...], preferred_element_type=_F32) + bini_ref[...]
    um = jnp.dot(u0, wm0_ref[...], preferred_element_type=_F32) + bm0_ref[...]
    for c in range(C):
        um_ref[c] = um[:, c * F:(c + 1) * F]
    b = batch_ref[...].reshape(1, NB)
    oh = (lax.broadcasted_iota(jnp.int32, (G, NB), 0) == b).astype(_F32)

    @pl.when(i == 0)
    def _():
        psumx_ref[...] = jnp.zeros_like(psumx_ref)
        counts_ref[...] = jnp.zeros_like(counts_ref)

    psumx_ref[...] += jnp.dot(oh, x, preferred_element_type=_F32)
    counts_ref[...] += jnp.broadcast_to(
        jnp.sum(oh, axis=1, keepdims=True), (G, H))


def _tc_init(x, w_init, b_init, wm0, bm0, batch3):
    return pl.pallas_call(
        _tc_init_body,
        grid=(NBLK,),
        in_specs=[
            pl.BlockSpec((NB, D_IN), lambda i: (i, 0)),
            pl.BlockSpec((D_IN, H), lambda i: (0, 0)),
            pl.BlockSpec((1, H), lambda i: (0, 0)),
            pl.BlockSpec((H, H), lambda i: (0, 0)),
            pl.BlockSpec((1, H), lambda i: (0, 0)),
            pl.BlockSpec((1, 1, NB), lambda i: (i, 0, 0)),
        ],
        out_specs=[
            pl.BlockSpec((C, NB, F), lambda i: (0, i, 0)),
            pl.BlockSpec((G, D_IN), lambda i: (0, 0)),
            pl.BlockSpec((G, H), lambda i: (0, 0)),
        ],
        out_shape=[
            jax.ShapeDtypeStruct((C, N, F), _F32),
            jax.ShapeDtypeStruct((G, D_IN), _F32),
            jax.ShapeDtypeStruct((G, H), _F32),
        ],
        compiler_params=pltpu.CompilerParams(
            dimension_semantics=("arbitrary",)),
    )(x, w_init, b_init, wm0, bm0, batch3)


def _tc_update_body(aggr_ref, um_ref, wi_ref, bi_ref, wj_ref, bj_ref,
                    wn_ref, bn_ref, batch_ref, umn_ref, psum_ref):
    i = pl.program_id(0)
    aggr = jnp.concatenate([aggr_ref[c] for c in range(C)], axis=-1)
    um = jnp.concatenate([um_ref[c] for c in range(C)], axis=-1)
    ai = um * wi_ref[...] + bi_ref[...]
    aj = aggr * wj_ref[...] + bj_ref[...]
    u = aggr + um + ai * aj
    umn = jnp.dot(u, wn_ref[...], preferred_element_type=_F32) + bn_ref[...]
    for c in range(C):
        umn_ref[c] = umn[:, c * F:(c + 1) * F]
    b = batch_ref[...].reshape(1, NB)
    oh = (lax.broadcasted_iota(jnp.int32, (G, NB), 0) == b).astype(_F32)

    @pl.when(i == 0)
    def _():
        psum_ref[...] = jnp.zeros_like(psum_ref)

    psum_ref[...] += jnp.dot(oh, u, preferred_element_type=_F32)


def _tc_update(aggr, um, wi, bi, wj, bj, wn, bn, batch3):
    return pl.pallas_call(
        _tc_update_body,
        grid=(NBLK,),
        in_specs=[
            pl.BlockSpec((C, NB, F), lambda i: (0, i, 0)),
            pl.BlockSpec((C, NB, F), lambda i: (0, i, 0)),
            pl.BlockSpec((1, H), lambda i: (0, 0)),
            pl.BlockSpec((1, H), lambda i: (0, 0)),
            pl.BlockSpec((1, H), lambda i: (0, 0)),
            pl.BlockSpec((1, H), lambda i: (0, 0)),
            pl.BlockSpec((H, H), lambda i: (0, 0)),
            pl.BlockSpec((1, H), lambda i: (0, 0)),
            pl.BlockSpec((1, 1, NB), lambda i: (i, 0, 0)),
        ],
        out_specs=[
            pl.BlockSpec((C, NB, F), lambda i: (0, i, 0)),
            pl.BlockSpec((G, H), lambda i: (0, 0)),
        ],
        out_shape=[
            jax.ShapeDtypeStruct((C, N, F), _F32),
            jax.ShapeDtypeStruct((G, H), _F32),
        ],
        compiler_params=pltpu.CompilerParams(
            dimension_semantics=("arbitrary",)),
    )(aggr, um, wi, bi, wj, bj, wn, bn, batch3)


def _tc_last_body(aggr_ref, um_ref, wi_ref, bi_ref, wj_ref, bj_ref,
                  batch_ref, psum_ref):
    i = pl.program_id(0)
    aggr = jnp.concatenate([aggr_ref[c] for c in range(C)], axis=-1)
    um = jnp.concatenate([um_ref[c] for c in range(C)], axis=-1)
    ai = um * wi_ref[...] + bi_ref[...]
    aj = aggr * wj_ref[...] + bj_ref[...]
    u = aggr + um + ai * aj
    b = batch_ref[...].reshape(1, NB)
    oh = (lax.broadcasted_iota(jnp.int32, (G, NB), 0) == b).astype(_F32)

    @pl.when(i == 0)
    def _():
        psum_ref[...] = jnp.zeros_like(psum_ref)

    psum_ref[...] += jnp.dot(oh, u, preferred_element_type=_F32)


def _tc_last(aggr, um, wi, bi, wj, bj, batch3):
    return pl.pallas_call(
        _tc_last_body,
        grid=(NBLK,),
        in_specs=[
            pl.BlockSpec((C, NB, F), lambda i: (0, i, 0)),
            pl.BlockSpec((C, NB, F), lambda i: (0, i, 0)),
            pl.BlockSpec((1, H), lambda i: (0, 0)),
            pl.BlockSpec((1, H), lambda i: (0, 0)),
            pl.BlockSpec((1, H), lambda i: (0, 0)),
            pl.BlockSpec((1, H), lambda i: (0, 0)),
            pl.BlockSpec((1, 1, NB), lambda i: (i, 0, 0)),
        ],
        out_specs=pl.BlockSpec((G, H), lambda i: (0, 0)),
        out_shape=jax.ShapeDtypeStruct((G, H), _F32),
        compiler_params=pltpu.CompilerParams(
            dimension_semantics=("arbitrary",)),
    )(aggr, um, wi, bi, wj, bj, batch3)


def _tc_head_body(psumx_ref, ps0_ref, ps1_ref, ps2_ref, ps3_ref, counts_ref,
                  wnp_ref, bnp_ref, we_ref, be_ref, wf_ref, bf_ref, out_ref):
    cnt = jnp.maximum(counts_ref[...], 1.0)
    poolx = psumx_ref[...] / cnt[:, :D_IN]
    out = jnp.dot(poolx, wnp_ref[...], preferred_element_type=_F32) + bnp_ref[...]
    for l, ps_ref in enumerate((ps0_ref, ps1_ref, ps2_ref, ps3_ref)):
        pool = ps_ref[...] / cnt
        out += (jnp.dot(pool, we_ref[l], preferred_element_type=_F32)
                + be_ref[l]) * (1.0 / NLAYERS)
    logits = jnp.dot(out, wf_ref[...], preferred_element_type=_F32) + bf_ref[...]
    m = jnp.max(logits, axis=-1, keepdims=True)
    s = logits - m
    lse = jnp.log(jnp.sum(jnp.exp(s), axis=-1, keepdims=True))
    out_ref[...] = s - lse


def _tc_head(psumx, psums, counts, w_np, b_np, we, be, wf, bf):
    return pl.pallas_call(
        _tc_head_body,
        out_shape=jax.ShapeDtypeStruct((G, NCLS), _F32),
    )(psumx, psums[0], psums[1], psums[2], psums[3], counts,
      w_np, b_np, we, be, wf, bf)


def _sc_aggregate(um_flat, srcoff, dst3, zrows):
    mesh = plsc.VectorSubcoreMesh(core_axis_name="c", subcore_axis_name="s")

    @functools.partial(
        pl.kernel,
        out_type=jax.ShapeDtypeStruct((C * N, F), _F32),
        mesh=mesh,
        scratch_types=[
            pltpu.VMEM((SRCBUF,), jnp.int32),
            pltpu.VMEM((NBAT, EB), jnp.int32),
            pltpu.VMEM((EB, F), _F32),
            pltpu.VMEM((EB, F), _F32),
            pltpu.VMEM_SHARED((NP, F), _F32),
            pltpu.SemaphoreType.DMA,
            pltpu.SemaphoreType.DMA,
            pltpu.SemaphoreType.DMA,
            pltpu.SemaphoreType.DMA,
        ],
    )
    def body(um_hbm, src_hbm, dst_hbm, z_hbm, out_hbm,
             src_v, dst_v, rows_a, rows_b, acc_sh, gsa, gsb, ssa, ssb):
        cid = lax.axis_index("c")
        sid = lax.axis_index("s")
        pltpu.sync_copy(dst_hbm.at[sid], dst_v)

        def lidx(l):
            return src_v.at[pl.ds(pl.multiple_of(l * EB, 8), EB)]

        def run_segment(chunk, seg_start, nb):
            pltpu.sync_copy(
                src_hbm.at[chunk, sid, pl.ds(seg_start * EB, nb * EB)],
                src_v.at[pl.ds(0, nb * EB)])
            pltpu.async_copy(um_hbm.at[lidx(0)], rows_a, gsa)

            def pair(i, carry):
                l = i * 2
                b = seg_start + l
                pltpu.make_async_copy(um_hbm.at[lidx(l)], rows_a, gsa).wait()

                @pl.when(l > 0)
                def _():
                    pltpu.make_async_copy(
                        rows_b, acc_sh.at[dst_v.at[b]], ssb).wait()

                pltpu.async_copy(um_hbm.at[lidx(l + 1)], rows_b, gsb)
                pltpu.async_copy(rows_a, acc_sh.at[dst_v.at[b]], ssa,
                                 add=True)
                pltpu.make_async_copy(
                    um_hbm.at[lidx(l + 1)], rows_b, gsb).wait()
                pltpu.make_async_copy(
                    rows_a, acc_sh.at[dst_v.at[b]], ssa).wait()

                @pl.when(l + 2 < nb)
                def _():
                    pltpu.async_copy(um_hbm.at[lidx(l + 2)], rows_a, gsa)

                pltpu.async_copy(rows_b, acc_sh.at[dst_v.at[b + 1]], ssb,
                                 add=True)
                return carry

            lax.fori_loop(0, nb // 2, pair, 0)
            if nb % 2 == 1:
                l = nb - 1
                b = seg_start + l
                pltpu.make_async_copy(um_hbm.at[lidx(l)], rows_a, gsa).wait()
                pltpu.make_async_copy(
                    rows_b, acc_sh.at[dst_v.at[b - 1]], ssb).wait()
                pltpu.async_copy(rows_a, acc_sh.at[dst_v.at[b]], ssa,
                                 add=True)
                pltpu.make_async_copy(
                    rows_a, acc_sh.at[dst_v.at[b]], ssa).wait()
            else:
                pltpu.make_async_copy(
                    rows_b, acc_sh.at[dst_v.at[seg_start]], ssb).wait()

        for c_local in range(NC):
            chunk = cid * NC + c_local

            for z in range(RB // ZR):
                pltpu.sync_copy(
                    z_hbm, acc_sh.at[pl.ds(sid * RB + z * ZR, ZR)])

            @pl.when(sid == 0)
            def _():
                pltpu.sync_copy(z_hbm.at[pl.ds(0, TAIL)],
                                acc_sh.at[pl.ds(NS * RB, TAIL)])

            plsc.subcore_barrier()

            for seg_start, nb in SEGS:
                run_segment(chunk, seg_start, nb)
            plsc.subcore_barrier()

            pltpu.sync_copy(
                acc_sh.at[pl.ds(sid * RB, RB)],
                out_hbm.at[pl.ds(chunk * N + sid * RB, RB)])

            @pl.when(sid == 0)
            def _():
                pltpu.sync_copy(
                    acc_sh.at[pl.ds(NS * RB, TAIL)],
                    out_hbm.at[pl.ds(chunk * N + NS * RB, TAIL)])

            plsc.subcore_barrier()

    return body(um_flat, srcoff, dst3, zrows)


def kernel(x, edge_index, batch, W_init, b_init, W_np, b_np,
           Wm, bm, wi, bi, wj, bj, We, be, Wf, bf):
    srcp = edge_index[0].reshape(NS, EPT)
    if PAD:
        srcp = jnp.concatenate(
            [srcp, jnp.zeros((NS, PAD), jnp.int32)], axis=1)
    srcoff = (srcp[None]
              + (jnp.arange(C, dtype=jnp.int32) * N)[:, None, None])
    dstp = edge_index[1].reshape(NS, EPT)
    if PAD:
        dump = N + jnp.broadcast_to(
            jnp.arange(NS, dtype=jnp.int32)[:, None], (NS, PAD))
        dstp = jnp.concatenate([dstp, dump], axis=1)
    dst3 = dstp.reshape(NS, NBAT, EB)
    batch3 = batch.reshape(NBLK, 1, NB)
    zrows = jnp.zeros((ZR, F), _F32)

    b_init2 = b_init.reshape(1, H)
    bm2 = bm.reshape(NLAYERS, 1, H)
    wi2 = wi.reshape(NLAYERS, 1, H)
    bi2 = bi.reshape(NLAYERS, 1, H)
    wj2 = wj.reshape(NLAYERS, 1, H)
    bj2 = bj.reshape(NLAYERS, 1, H)
    be2 = be.reshape(NLAYERS, 1, HF)
    b_np2 = b_np.reshape(1, HF)
    bf2 = bf.reshape(1, NCLS)

    um, psumx, counts = _tc_init(x, W_init, b_init2, Wm[0], bm2[0], batch3)
    psums = []
    for l in range(NLAYERS):
        aggr = _sc_aggregate(um.reshape(C * N, F), srcoff, dst3, zrows)
        aggr = aggr.reshape(C, N, F)
        if l < NLAYERS - 1:
            um, ps = _tc_update(aggr, um, wi2[l], bi2[l], wj2[l], bj2[l],
                                Wm[l + 1], bm2[l + 1], batch3)
        else:
            ps = _tc_last(aggr, um, wi2[l], bi2[l], wj2[l], bj2[l], batch3)
        psums.append(ps)

    return _tc_head(psumx, psums, counts, W_np, b_np2, We, be2, Wf, bf2)


# speedup vs baseline: 1.2240x; 1.0059x over previous
import functools

import jax
import jax.numpy as jnp
from jax import lax
from jax.experimental import pallas as pl
from jax.experimental.pallas import tpu as pltpu
from jax.experimental.pallas import tpu_sc as plsc

N = 10000
E = 160000
D_IN = 256
H = 512
HF = 256
NCLS = 10
NLAYERS = 4
G = 16

NB = 1000
NBLK = N // NB

C = 4
F = H // C
NC = 2
NS = 16
EPT = E // NS
EB = 80
NBAT = EPT // EB
EPAD = NBAT * EB
PAD = EPAD - EPT
NP = N
SEGS = ((0, NBAT),)
SRCBUF = NBAT * EB
RB = 624
TAIL = N - NS * RB
ZR = 104

_F32 = jnp.float32


def _tc_init_body(x_ref, wini_ref, bini_ref, wm0_ref, bm0_ref, batch_ref,
                  um_ref, psumx_ref, counts_ref):
    i = pl.program_id(0)
    x = x_ref[...]
    u0 = jnp.dot(x.astype(jnp.bfloat16), wini_ref[...],
                 preferred_element_type=_F32) + bini_ref[...]
    um = jnp.dot(u0.astype(jnp.bfloat16), wm0_ref[...],
                 preferred_element_type=_F32) + bm0_ref[...]
    for c in range(C):
        um_ref[c] = um[:, c * F:(c + 1) * F]
    b = batch_ref[...].reshape(1, NB)
    oh = (lax.broadcasted_iota(jnp.int32, (G, NB), 0) == b).astype(_F32)

    @pl.when(i == 0)
    def _():
        psumx_ref[...] = jnp.zeros_like(psumx_ref)
        counts_ref[...] = jnp.zeros_like(counts_ref)

    psumx_ref[...] += jnp.dot(oh, x, preferred_element_type=_F32)
    counts_ref[...] += jnp.broadcast_to(
        jnp.sum(oh, axis=1, keepdims=True), (G, H))


def _tc_init(x, w_init, b_init, wm0, bm0, batch3):
    return pl.pallas_call(
        _tc_init_body,
        grid=(NBLK,),
        in_specs=[
            pl.BlockSpec((NB, D_IN), lambda i: (i, 0)),
            pl.BlockSpec((D_IN, H), lambda i: (0, 0)),
            pl.BlockSpec((1, H), lambda i: (0, 0)),
            pl.BlockSpec((H, H), lambda i: (0, 0)),
            pl.BlockSpec((1, H), lambda i: (0, 0)),
            pl.BlockSpec((1, 1, NB), lambda i: (i, 0, 0)),
        ],
        out_specs=[
            pl.BlockSpec((C, NB, F), lambda i: (0, i, 0)),
            pl.BlockSpec((G, D_IN), lambda i: (0, 0)),
            pl.BlockSpec((G, H), lambda i: (0, 0)),
        ],
        out_shape=[
            jax.ShapeDtypeStruct((C, N, F), _F32),
            jax.ShapeDtypeStruct((G, D_IN), _F32),
            jax.ShapeDtypeStruct((G, H), _F32),
        ],
        compiler_params=pltpu.CompilerParams(
            dimension_semantics=("arbitrary",)),
    )(x, w_init, b_init, wm0, bm0, batch3)


def _tc_update_body(aggr_ref, um_ref, wi_ref, bi_ref, wj_ref, bj_ref,
                    wn_ref, bn_ref, batch_ref, umn_ref, psum_ref):
    i = pl.program_id(0)
    aggr = jnp.concatenate([aggr_ref[c] for c in range(C)], axis=-1)
    um = jnp.concatenate([um_ref[c] for c in range(C)], axis=-1)
    ai = um * wi_ref[...] + bi_ref[...]
    aj = aggr * wj_ref[...] + bj_ref[...]
    u = aggr + um + ai * aj
    umn = jnp.dot(u.astype(jnp.bfloat16), wn_ref[...],
                  preferred_element_type=_F32) + bn_ref[...]
    for c in range(C):
        umn_ref[c] = umn[:, c * F:(c + 1) * F]
    b = batch_ref[...].reshape(1, NB)
    oh = (lax.broadcasted_iota(jnp.int32, (G, NB), 0) == b).astype(_F32)

    @pl.when(i == 0)
    def _():
        psum_ref[...] = jnp.zeros_like(psum_ref)

    psum_ref[...] += jnp.dot(oh, u, preferred_element_type=_F32)


def _tc_update(aggr, um, wi, bi, wj, bj, wn, bn, batch3):
    return pl.pallas_call(
        _tc_update_body,
        grid=(NBLK,),
        in_specs=[
            pl.BlockSpec((C, NB, F), lambda i: (0, i, 0)),
            pl.BlockSpec((C, NB, F), lambda i: (0, i, 0)),
            pl.BlockSpec((1, H), lambda i: (0, 0)),
            pl.BlockSpec((1, H), lambda i: (0, 0)),
            pl.BlockSpec((1, H), lambda i: (0, 0)),
            pl.BlockSpec((1, H), lambda i: (0, 0)),
            pl.BlockSpec((H, H), lambda i: (0, 0)),
            pl.BlockSpec((1, H), lambda i: (0, 0)),
            pl.BlockSpec((1, 1, NB), lambda i: (i, 0, 0)),
        ],
        out_specs=[
            pl.BlockSpec((C, NB, F), lambda i: (0, i, 0)),
            pl.BlockSpec((G, H), lambda i: (0, 0)),
        ],
        out_shape=[
            jax.ShapeDtypeStruct((C, N, F), _F32),
            jax.ShapeDtypeStruct((G, H), _F32),
        ],
        compiler_params=pltpu.CompilerParams(
            dimension_semantics=("arbitrary",)),
    )(aggr, um, wi, bi, wj, bj, wn, bn, batch3)


def _tc_last_body(aggr_ref, um_ref, wi_ref, bi_ref, wj_ref, bj_ref,
                  batch_ref, psum_ref):
    i = pl.program_id(0)
    aggr = jnp.concatenate([aggr_ref[c] for c in range(C)], axis=-1)
    um = jnp.concatenate([um_ref[c] for c in range(C)], axis=-1)
    ai = um * wi_ref[...] + bi_ref[...]
    aj = aggr * wj_ref[...] + bj_ref[...]
    u = aggr + um + ai * aj
    b = batch_ref[...].reshape(1, NB)
    oh = (lax.broadcasted_iota(jnp.int32, (G, NB), 0) == b).astype(_F32)

    @pl.when(i == 0)
    def _():
        psum_ref[...] = jnp.zeros_like(psum_ref)

    psum_ref[...] += jnp.dot(oh, u, preferred_element_type=_F32)


def _tc_last(aggr, um, wi, bi, wj, bj, batch3):
    return pl.pallas_call(
        _tc_last_body,
        grid=(NBLK,),
        in_specs=[
            pl.BlockSpec((C, NB, F), lambda i: (0, i, 0)),
            pl.BlockSpec((C, NB, F), lambda i: (0, i, 0)),
            pl.BlockSpec((1, H), lambda i: (0, 0)),
            pl.BlockSpec((1, H), lambda i: (0, 0)),
            pl.BlockSpec((1, H), lambda i: (0, 0)),
            pl.BlockSpec((1, H), lambda i: (0, 0)),
            pl.BlockSpec((1, 1, NB), lambda i: (i, 0, 0)),
        ],
        out_specs=pl.BlockSpec((G, H), lambda i: (0, 0)),
        out_shape=jax.ShapeDtypeStruct((G, H), _F32),
        compiler_params=pltpu.CompilerParams(
            dimension_semantics=("arbitrary",)),
    )(aggr, um, wi, bi, wj, bj, batch3)


def _tc_head_body(psumx_ref, ps0_ref, ps1_ref, ps2_ref, ps3_ref, counts_ref,
                  wnp_ref, bnp_ref, we_ref, be_ref, wf_ref, bf_ref, out_ref):
    cnt = jnp.maximum(counts_ref[...], 1.0)
    poolx = psumx_ref[...] / cnt[:, :D_IN]
    out = jnp.dot(poolx, wnp_ref[...], preferred_element_type=_F32) + bnp_ref[...]
    for l, ps_ref in enumerate((ps0_ref, ps1_ref, ps2_ref, ps3_ref)):
        pool = ps_ref[...] / cnt
        out += (jnp.dot(pool, we_ref[l], preferred_element_type=_F32)
                + be_ref[l]) * (1.0 / NLAYERS)
    logits = jnp.dot(out, wf_ref[...], preferred_element_type=_F32) + bf_ref[...]
    m = jnp.max(logits, axis=-1, keepdims=True)
    s = logits - m
    lse = jnp.log(jnp.sum(jnp.exp(s), axis=-1, keepdims=True))
    out_ref[...] = s - lse


def _tc_head(psumx, psums, counts, w_np, b_np, we, be, wf, bf):
    return pl.pallas_call(
        _tc_head_body,
        out_shape=jax.ShapeDtypeStruct((G, NCLS), _F32),
    )(psumx, psums[0], psums[1], psums[2], psums[3], counts,
      w_np, b_np, we, be, wf, bf)


def _sc_aggregate(um_flat, srcoff, dst3, zrows):
    mesh = plsc.VectorSubcoreMesh(core_axis_name="c", subcore_axis_name="s")

    @functools.partial(
        pl.kernel,
        out_type=jax.ShapeDtypeStruct((C * N, F), _F32),
        mesh=mesh,
        scratch_types=[
            pltpu.VMEM((SRCBUF,), jnp.int32),
            pltpu.VMEM((NBAT, EB), jnp.int32),
            pltpu.VMEM((EB, F), _F32),
            pltpu.VMEM((EB, F), _F32),
            pltpu.VMEM_SHARED((NP, F), _F32),
            pltpu.SemaphoreType.DMA,
            pltpu.SemaphoreType.DMA,
            pltpu.SemaphoreType.DMA,
            pltpu.SemaphoreType.DMA,
        ],
    )
    def body(um_hbm, src_hbm, dst_hbm, z_hbm, out_hbm,
             src_v, dst_v, rows_a, rows_b, acc_sh, gsa, gsb, ssa, ssb):
        cid = lax.axis_index("c")
        sid = lax.axis_index("s")
        pltpu.sync_copy(dst_hbm.at[sid], dst_v)

        def lidx(l):
            return src_v.at[pl.ds(pl.multiple_of(l * EB, 8), EB)]

        def run_segment(chunk, seg_start, nb):
            pltpu.sync_copy(
                src_hbm.at[chunk, sid, pl.ds(seg_start * EB, nb * EB)],
                src_v.at[pl.ds(0, nb * EB)])
            pltpu.async_copy(um_hbm.at[lidx(0)], rows_a, gsa)

            def pair(i, carry):
                l = i * 2
                b = seg_start + l
                pltpu.make_async_copy(um_hbm.at[lidx(l)], rows_a, gsa).wait()

                @pl.when(l > 0)
                def _():
                    pltpu.make_async_copy(
                        rows_b, acc_sh.at[dst_v.at[b]], ssb).wait()

                pltpu.async_copy(um_hbm.at[lidx(l + 1)], rows_b, gsb)
                pltpu.async_copy(rows_a, acc_sh.at[dst_v.at[b]], ssa,
                                 add=True)
                pltpu.make_async_copy(
                    um_hbm.at[lidx(l + 1)], rows_b, gsb).wait()
                pltpu.make_async_copy(
                    rows_a, acc_sh.at[dst_v.at[b]], ssa).wait()

                @pl.when(l + 2 < nb)
                def _():
                    pltpu.async_copy(um_hbm.at[lidx(l + 2)], rows_a, gsa)

                pltpu.async_copy(rows_b, acc_sh.at[dst_v.at[b + 1]], ssb,
                                 add=True)
                return carry

            lax.fori_loop(0, nb // 2, pair, 0)
            if nb % 2 == 1:
                l = nb - 1
                b = seg_start + l
                pltpu.make_async_copy(um_hbm.at[lidx(l)], rows_a, gsa).wait()
                pltpu.make_async_copy(
                    rows_b, acc_sh.at[dst_v.at[b - 1]], ssb).wait()
                pltpu.async_copy(rows_a, acc_sh.at[dst_v.at[b]], ssa,
                                 add=True)
                pltpu.make_async_copy(
                    rows_a, acc_sh.at[dst_v.at[b]], ssa).wait()
            else:
                pltpu.make_async_copy(
                    rows_b, acc_sh.at[dst_v.at[seg_start]], ssb).wait()

        for c_local in range(NC):
            chunk = cid * NC + c_local

            for z in range(RB // ZR):
                pltpu.sync_copy(
                    z_hbm, acc_sh.at[pl.ds(sid * RB + z * ZR, ZR)])

            @pl.when(sid == 0)
            def _():
                pltpu.sync_copy(z_hbm.at[pl.ds(0, TAIL)],
                                acc_sh.at[pl.ds(NS * RB, TAIL)])

            plsc.subcore_barrier()

            for seg_start, nb in SEGS:
                run_segment(chunk, seg_start, nb)
            plsc.subcore_barrier()

            pltpu.sync_copy(
                acc_sh.at[pl.ds(sid * RB, RB)],
                out_hbm.at[pl.ds(chunk * N + sid * RB, RB)])

            @pl.when(sid == 0)
            def _():
                pltpu.sync_copy(
                    acc_sh.at[pl.ds(NS * RB, TAIL)],
                    out_hbm.at[pl.ds(chunk * N + NS * RB, TAIL)])

            plsc.subcore_barrier()

    return body(um_flat, srcoff, dst3, zrows)


def kernel(x, edge_index, batch, W_init, b_init, W_np, b_np,
           Wm, bm, wi, bi, wj, bj, We, be, Wf, bf):
    srcp = edge_index[0].reshape(NS, EPT)
    if PAD:
        srcp = jnp.concatenate(
            [srcp, jnp.zeros((NS, PAD), jnp.int32)], axis=1)
    srcoff = (srcp[None]
              + (jnp.arange(C, dtype=jnp.int32) * N)[:, None, None])
    dstp = edge_index[1].reshape(NS, EPT)
    if PAD:
        dump = N + jnp.broadcast_to(
            jnp.arange(NS, dtype=jnp.int32)[:, None], (NS, PAD))
        dstp = jnp.concatenate([dstp, dump], axis=1)
    dst3 = dstp.reshape(NS, NBAT, EB)
    batch3 = batch.reshape(NBLK, 1, NB)
    zrows = jnp.zeros((ZR, F), _F32)

    b_init2 = b_init.reshape(1, H)
    bm2 = bm.reshape(NLAYERS, 1, H)
    wi2 = wi.reshape(NLAYERS, 1, H)
    bi2 = bi.reshape(NLAYERS, 1, H)
    wj2 = wj.reshape(NLAYERS, 1, H)
    bj2 = bj.reshape(NLAYERS, 1, H)
    be2 = be.reshape(NLAYERS, 1, HF)
    b_np2 = b_np.reshape(1, HF)
    bf2 = bf.reshape(1, NCLS)

    W_init16 = W_init.astype(jnp.bfloat16)
    Wm16 = Wm.astype(jnp.bfloat16)

    um, psumx, counts = _tc_init(x, W_init16, b_init2, Wm16[0], bm2[0],
                                 batch3)
    psums = []
    for l in range(NLAYERS):
        aggr = _sc_aggregate(um.reshape(C * N, F), srcoff, dst3, zrows)
        aggr = aggr.reshape(C, N, F)
        if l < NLAYERS - 1:
            um, ps = _tc_update(aggr, um, wi2[l], bi2[l], wj2[l], bj2[l],
                                Wm16[l + 1], bm2[l + 1], batch3)
        else:
            ps = _tc_last(aggr, um, wi2[l], bi2[l], wj2[l], bj2[l], batch3)
        psums.append(ps)

    return _tc_head(psumx, psums, counts, W_np, b_np2, We, be2, Wf, bf2)


# speedup vs baseline: 1.7169x; 1.4027x over previous
import functools

import jax
import jax.numpy as jnp
from jax import lax
from jax.experimental import pallas as pl
from jax.experimental.pallas import tpu as pltpu
from jax.experimental.pallas import tpu_sc as plsc

N = 10000
E = 160000
D_IN = 256
H = 512
HF = 256
NCLS = 10
NLAYERS = 4
G = 16

NB = 1000
NBLK = N // NB

C = 4
F = H // C
NC = 2
NS = 16
EPT = E // NS
EB = 80
NBAT = EPT // EB
EPAD = NBAT * EB
PAD = EPAD - EPT
NP = N
SEGS = ((0, 48), (48, 48), (96, 29))
SRCBUF = 48 * EB
RB = 624
TAIL = N - NS * RB
ZR = 104

_F32 = jnp.float32


def _tc_init_body(x_ref, wini_ref, bini_ref, wm0_ref, bm0_ref, batch_ref,
                  um_ref, psumx_ref, counts_ref):
    i = pl.program_id(0)
    x = x_ref[...]
    u0 = jnp.dot(x.astype(jnp.bfloat16), wini_ref[...],
                 preferred_element_type=_F32) + bini_ref[...]
    um = jnp.dot(u0.astype(jnp.bfloat16), wm0_ref[...],
                 preferred_element_type=_F32) + bm0_ref[...]
    for c in range(C):
        um_ref[c] = um[:, c * F:(c + 1) * F]
    b = batch_ref[...].reshape(1, NB)
    oh = (lax.broadcasted_iota(jnp.int32, (G, NB), 0) == b).astype(_F32)

    @pl.when(i == 0)
    def _():
        psumx_ref[...] = jnp.zeros_like(psumx_ref)
        counts_ref[...] = jnp.zeros_like(counts_ref)

    psumx_ref[...] += jnp.dot(oh, x, preferred_element_type=_F32)
    counts_ref[...] += jnp.broadcast_to(
        jnp.sum(oh, axis=1, keepdims=True), (G, H))


def _tc_init(x, w_init, b_init, wm0, bm0, batch3):
    return pl.pallas_call(
        _tc_init_body,
        grid=(NBLK,),
        in_specs=[
            pl.BlockSpec((NB, D_IN), lambda i: (i, 0)),
            pl.BlockSpec((D_IN, H), lambda i: (0, 0)),
            pl.BlockSpec((1, H), lambda i: (0, 0)),
            pl.BlockSpec((H, H), lambda i: (0, 0)),
            pl.BlockSpec((1, H), lambda i: (0, 0)),
            pl.BlockSpec((1, 1, NB), lambda i: (i, 0, 0)),
        ],
        out_specs=[
            pl.BlockSpec((C, NB, F), lambda i: (0, i, 0)),
            pl.BlockSpec((G, D_IN), lambda i: (0, 0)),
            pl.BlockSpec((G, H), lambda i: (0, 0)),
        ],
        out_shape=[
            jax.ShapeDtypeStruct((C, N, F), _F32),
            jax.ShapeDtypeStruct((G, D_IN), _F32),
            jax.ShapeDtypeStruct((G, H), _F32),
        ],
        compiler_params=pltpu.CompilerParams(
            dimension_semantics=("arbitrary",)),
    )(x, w_init, b_init, wm0, bm0, batch3)


def _tc_update_body(aggr_ref, um_ref, wi_ref, bi_ref, wj_ref, bj_ref,
                    wn_ref, bn_ref, batch_ref, umn_ref, psum_ref):
    i = pl.program_id(0)
    aggr = jnp.concatenate([aggr_ref[c] for c in range(C)], axis=-1)
    um = jnp.concatenate([um_ref[c] for c in range(C)], axis=-1)
    ai = um * wi_ref[...] + bi_ref[...]
    aj = aggr * wj_ref[...] + bj_ref[...]
    u = aggr + um + ai * aj
    umn = jnp.dot(u.astype(jnp.bfloat16), wn_ref[...],
                  preferred_element_type=_F32) + bn_ref[...]
    for c in range(C):
        umn_ref[c] = umn[:, c * F:(c + 1) * F]
    b = batch_ref[...].reshape(1, NB)
    oh = (lax.broadcasted_iota(jnp.int32, (G, NB), 0) == b).astype(_F32)

    @pl.when(i == 0)
    def _():
        psum_ref[...] = jnp.zeros_like(psum_ref)

    psum_ref[...] += jnp.dot(oh, u, preferred_element_type=_F32)


def _tc_update(aggr, um, wi, bi, wj, bj, wn, bn, batch3):
    return pl.pallas_call(
        _tc_update_body,
        grid=(NBLK,),
        in_specs=[
            pl.BlockSpec((C, NB, F), lambda i: (0, i, 0)),
            pl.BlockSpec((C, NB, F), lambda i: (0, i, 0)),
            pl.BlockSpec((1, H), lambda i: (0, 0)),
            pl.BlockSpec((1, H), lambda i: (0, 0)),
            pl.BlockSpec((1, H), lambda i: (0, 0)),
            pl.BlockSpec((1, H), lambda i: (0, 0)),
            pl.BlockSpec((H, H), lambda i: (0, 0)),
            pl.BlockSpec((1, H), lambda i: (0, 0)),
            pl.BlockSpec((1, 1, NB), lambda i: (i, 0, 0)),
        ],
        out_specs=[
            pl.BlockSpec((C, NB, F), lambda i: (0, i, 0)),
            pl.BlockSpec((G, H), lambda i: (0, 0)),
        ],
        out_shape=[
            jax.ShapeDtypeStruct((C, N, F), _F32),
            jax.ShapeDtypeStruct((G, H), _F32),
        ],
        compiler_params=pltpu.CompilerParams(
            dimension_semantics=("arbitrary",)),
    )(aggr, um, wi, bi, wj, bj, wn, bn, batch3)


def _tc_last_body(aggr_ref, um_ref, wi_ref, bi_ref, wj_ref, bj_ref,
                  batch_ref, psum_ref):
    i = pl.program_id(0)
    aggr = jnp.concatenate([aggr_ref[c] for c in range(C)], axis=-1)
    um = jnp.concatenate([um_ref[c] for c in range(C)], axis=-1)
    ai = um * wi_ref[...] + bi_ref[...]
    aj = aggr * wj_ref[...] + bj_ref[...]
    u = aggr + um + ai * aj
    b = batch_ref[...].reshape(1, NB)
    oh = (lax.broadcasted_iota(jnp.int32, (G, NB), 0) == b).astype(_F32)

    @pl.when(i == 0)
    def _():
        psum_ref[...] = jnp.zeros_like(psum_ref)

    psum_ref[...] += jnp.dot(oh, u, preferred_element_type=_F32)


def _tc_last(aggr, um, wi, bi, wj, bj, batch3):
    return pl.pallas_call(
        _tc_last_body,
        grid=(NBLK,),
        in_specs=[
            pl.BlockSpec((C, NB, F), lambda i: (0, i, 0)),
            pl.BlockSpec((C, NB, F), lambda i: (0, i, 0)),
            pl.BlockSpec((1, H), lambda i: (0, 0)),
            pl.BlockSpec((1, H), lambda i: (0, 0)),
            pl.BlockSpec((1, H), lambda i: (0, 0)),
            pl.BlockSpec((1, H), lambda i: (0, 0)),
            pl.BlockSpec((1, 1, NB), lambda i: (i, 0, 0)),
        ],
        out_specs=pl.BlockSpec((G, H), lambda i: (0, 0)),
        out_shape=jax.ShapeDtypeStruct((G, H), _F32),
        compiler_params=pltpu.CompilerParams(
            dimension_semantics=("arbitrary",)),
    )(aggr, um, wi, bi, wj, bj, batch3)


def _tc_head_body(psumx_ref, ps0_ref, ps1_ref, ps2_ref, ps3_ref, counts_ref,
                  wnp_ref, bnp_ref, we_ref, be_ref, wf_ref, bf_ref, out_ref):
    cnt = jnp.maximum(counts_ref[...], 1.0)
    poolx = psumx_ref[...] / cnt[:, :D_IN]
    out = jnp.dot(poolx, wnp_ref[...], preferred_element_type=_F32) + bnp_ref[...]
    for l, ps_ref in enumerate((ps0_ref, ps1_ref, ps2_ref, ps3_ref)):
        pool = ps_ref[...] / cnt
        out += (jnp.dot(pool, we_ref[l], preferred_element_type=_F32)
                + be_ref[l]) * (1.0 / NLAYERS)
    logits = jnp.dot(out, wf_ref[...], preferred_element_type=_F32) + bf_ref[...]
    m = jnp.max(logits, axis=-1, keepdims=True)
    s = logits - m
    lse = jnp.log(jnp.sum(jnp.exp(s), axis=-1, keepdims=True))
    out_ref[...] = s - lse


def _tc_head(psumx, psums, counts, w_np, b_np, we, be, wf, bf):
    return pl.pallas_call(
        _tc_head_body,
        out_shape=jax.ShapeDtypeStruct((G, NCLS), _F32),
    )(psumx, psums[0], psums[1], psums[2], psums[3], counts,
      w_np, b_np, we, be, wf, bf)


def _sc_aggregate(um_flat, srcoff, dst3, zrows):
    mesh = plsc.VectorSubcoreMesh(core_axis_name="c", subcore_axis_name="s")

    @functools.partial(
        pl.kernel,
        out_type=jax.ShapeDtypeStruct((C * N, F), _F32),
        mesh=mesh,
        scratch_types=[
            pltpu.VMEM((SRCBUF,), jnp.int32),
            pltpu.VMEM((NBAT, EB), jnp.int32),
            pltpu.VMEM((EB, F), _F32),
            pltpu.VMEM((EB, F), _F32),
            pltpu.VMEM((EB, F), _F32),
            pltpu.VMEM_SHARED((NP, F), _F32),
            pltpu.SemaphoreType.DMA,
            pltpu.SemaphoreType.DMA,
            pltpu.SemaphoreType.DMA,
            pltpu.SemaphoreType.DMA,
            pltpu.SemaphoreType.DMA,
            pltpu.SemaphoreType.DMA,
        ],
    )
    def body(um_hbm, src_hbm, dst_hbm, z_hbm, out_hbm,
             src_v, dst_v, r0, r1, r2, acc_sh, g0, g1, g2, s0, s1, s2):
        cid = lax.axis_index("c")
        sid = lax.axis_index("s")
        pltpu.sync_copy(dst_hbm.at[sid], dst_v)

        rows = (r0, r1, r2)
        gsem = (g0, g1, g2)
        ssem = (s0, s1, s2)

        def lidx(l):
            return src_v.at[pl.ds(pl.multiple_of(l * EB, 8), EB)]

        def gwait(l, j):
            pltpu.make_async_copy(um_hbm.at[lidx(l)], rows[j], gsem[j]).wait()

        def swait(b, j):
            pltpu.make_async_copy(
                rows[j], acc_sh.at[dst_v.at[b]], ssem[j]).wait()

        def run_segment(chunk, seg_start, nb):
            base = (chunk * NS + sid) * EPAD + seg_start * EB
            pltpu.sync_copy(
                src_hbm.at[pl.ds(pl.multiple_of(base, 8), nb * EB)],
                src_v.at[pl.ds(0, nb * EB)])
            pltpu.async_copy(um_hbm.at[lidx(0)], rows[0], gsem[0])
            pltpu.async_copy(um_hbm.at[lidx(1)], rows[1], gsem[1])

            def visit(l, j, skip_swait=False, traced_guard=None):
                gwait(l, j)
                pltpu.async_copy(rows[j], acc_sh.at[dst_v.at[seg_start + l]],
                                 ssem[j], add=True)
                j2 = (j + 2) % 3

                def prefetch():
                    if not skip_swait:
                        swait(seg_start + l, j2)
                    pltpu.async_copy(um_hbm.at[lidx(l + 2)], rows[j2],
                                     gsem[j2])

                if traced_guard is not None:
                    pl.when(traced_guard)(prefetch)
                elif isinstance(l, int) and l + 2 < nb:
                    prefetch()

            visit(0, 0, skip_swait=True)
            visit(1, 1)
            visit(2, 2)

            ntrip = nb // 3

            def triple(i, carry):
                base = i * 3
                for j in range(3):
                    l = base + j
                    visit(l, j, traced_guard=l + 2 < nb)
                return carry

            lax.fori_loop(1, ntrip, triple, 0)
            for t in range(nb % 3):
                l = ntrip * 3 + t
                if l > 2:
                    visit(l, t)
            swait(seg_start + nb - 3, (nb - 3) % 3)
            swait(seg_start + nb - 2, (nb - 2) % 3)
            swait(seg_start + nb - 1, (nb - 1) % 3)

        for c_local in range(NC):
            chunk = cid * NC + c_local

            for z in range(RB // ZR):
                pltpu.sync_copy(
                    z_hbm, acc_sh.at[pl.ds(sid * RB + z * ZR, ZR)])

            @pl.when(sid == 0)
            def _():
                pltpu.sync_copy(z_hbm.at[pl.ds(0, TAIL)],
                                acc_sh.at[pl.ds(NS * RB, TAIL)])

            plsc.subcore_barrier()

            for seg_start, nb in SEGS:
                run_segment(chunk, seg_start, nb)
            plsc.subcore_barrier()

            pltpu.sync_copy(
                acc_sh.at[pl.ds(sid * RB, RB)],
                out_hbm.at[pl.ds(chunk * N + sid * RB, RB)])

            @pl.when(sid == 0)
            def _():
                pltpu.sync_copy(
                    acc_sh.at[pl.ds(NS * RB, TAIL)],
                    out_hbm.at[pl.ds(chunk * N + NS * RB, TAIL)])

            plsc.subcore_barrier()

    return body(um_flat, srcoff, dst3, zrows)


def kernel(x, edge_index, batch, W_init, b_init, W_np, b_np,
           Wm, bm, wi, bi, wj, bj, We, be, Wf, bf):
    srcp = edge_index[0].reshape(NS, EPT)
    if PAD:
        srcp = jnp.concatenate(
            [srcp, jnp.zeros((NS, PAD), jnp.int32)], axis=1)
    srcoff = (srcp[None]
              + (jnp.arange(C, dtype=jnp.int32) * N)[:, None, None])
    dstp = edge_index[1].reshape(NS, EPT)
    if PAD:
        dump = N + jnp.broadcast_to(
            jnp.arange(NS, dtype=jnp.int32)[:, None], (NS, PAD))
        dstp = jnp.concatenate([dstp, dump], axis=1)
    dst3 = dstp.reshape(NS, NBAT, EB)
    batch3 = batch.reshape(NBLK, 1, NB)
    zrows = jnp.zeros((ZR, F), _F32)

    b_init2 = b_init.reshape(1, H)
    bm2 = bm.reshape(NLAYERS, 1, H)
    wi2 = wi.reshape(NLAYERS, 1, H)
    bi2 = bi.reshape(NLAYERS, 1, H)
    wj2 = wj.reshape(NLAYERS, 1, H)
    bj2 = bj.reshape(NLAYERS, 1, H)
    be2 = be.reshape(NLAYERS, 1, HF)
    b_np2 = b_np.reshape(1, HF)
    bf2 = bf.reshape(1, NCLS)

    W_init16 = W_init.astype(jnp.bfloat16)
    Wm16 = Wm.astype(jnp.bfloat16)

    um, psumx, counts = _tc_init(x, W_init16, b_init2, Wm16[0], bm2[0],
                                 batch3)
    psums = []
    for l in range(NLAYERS):
        aggr = _sc_aggregate(um.reshape(C * N, F), srcoff.reshape(-1),
                             dst3, zrows)
        aggr = aggr.reshape(C, N, F)
        if l < NLAYERS - 1:
            um, ps = _tc_update(aggr, um, wi2[l], bi2[l], wj2[l], bj2[l],
                                Wm16[l + 1], bm2[l + 1], batch3)
        else:
            ps = _tc_last(aggr, um, wi2[l], bi2[l], wj2[l], bj2[l], batch3)
        psums.append(ps)

    return _tc_head(psumx, psums, counts, W_np, b_np2, We, be2, Wf, bf2)
